# Initial kernel scaffold; baseline (speedup 1.0000x reference)
import jax
import jax.numpy as jnp
from jax.experimental import pallas as pl


def kernel(h, ln1_w, ln1_b, fc_W, head_W, head_b, ln2_w, ln2_b, si_W1, si_b1, si_W2, si_b2, edge_index):
    raise NotImplementedError("write your pallas kernel here")


# speedup vs baseline: 49.1421x; 49.1421x over previous
import functools

import jax
import jax.numpy as jnp
import numpy as np
from jax import lax
from jax.experimental import pallas as pl
from jax.experimental.pallas import tpu as pltpu
from jax.experimental.pallas import tpu_sc as plsc

_N = 10000
_E = 320000
_D = 128
_H = 8
_DH = 16

_NC = 2
_NS = 16
_NW = _NC * _NS
_EW = _E // _NW
_G = 80
_NG = _EW // _G
_NP = 10240
_RPS = _NP // _NS
_AW = 144


def _pre_body(h_ref, w_ref, b_ref, fc_ref, hn_ref, ft_ref):
    x = h_ref[...]
    mu = jnp.mean(x, axis=1, keepdims=True)
    var = jnp.mean((x - mu) ** 2, axis=1, keepdims=True)
    hn = (x - mu) * lax.rsqrt(var + 1e-5) * w_ref[...] + b_ref[...]
    hn_ref[...] = hn
    ft_ref[...] = 0.5 * jnp.dot(hn, fc_ref[...],
                                preferred_element_type=jnp.float32)


def _pre(h, ln1_w, ln1_b, fc_W):
    blk = 1000
    grid = (_N // blk,)
    return pl.pallas_call(
        _pre_body,
        grid=grid,
        in_specs=[
            pl.BlockSpec((blk, _D), lambda i: (i, 0)),
            pl.BlockSpec((1, _D), lambda i: (0, 0)),
            pl.BlockSpec((1, _D), lambda i: (0, 0)),
            pl.BlockSpec((_D, _D), lambda i: (0, 0)),
        ],
        out_specs=[
            pl.BlockSpec((blk, _D), lambda i: (i, 0)),
            pl.BlockSpec((blk, _D), lambda i: (i, 0)),
        ],
        out_shape=[
            jax.ShapeDtypeStruct((_N, _D), jnp.float32),
            jax.ShapeDtypeStruct((_N, _D), jnp.float32),
        ],
    )(h, ln1_w.reshape(1, _D), ln1_b.reshape(1, _D), fc_W)


def _lane_bcast(v, lane):
    idx = jnp.full((16, 1), lane, jnp.int32)
    dn = lax.GatherDimensionNumbers(offset_dims=(), collapsed_slice_dims=(0,),
                                    start_index_map=(0,))
    return lax.gather(v, idx, dn, (1,),
                      mode=lax.GatherScatterMode.PROMISE_IN_BOUNDS)


def _edge_body(ft_hbm, src_hbm, dst_hbm, zeros_hbm, acc_hbm,
               acc_sp, sidx, didx, srows, drows, wrows):
    c = lax.axis_index("c")
    s = lax.axis_index("s")
    rbase = s * _RPS
    pltpu.sync_copy(zeros_hbm.at[pl.ds(rbase, _RPS)],
                    acc_sp.at[pl.ds(rbase, _RPS)])
    plsc.subcore_barrier()

    wid = c * _NS + s
    ebase = wid * _EW

    def group(g, carry):
        gb = ebase + g * _G
        pltpu.sync_copy(src_hbm.at[pl.ds(gb, _G)], sidx)
        pltpu.sync_copy(dst_hbm.at[pl.ds(gb, _G)], didx)
        pltpu.sync_copy(ft_hbm.at[sidx], srows)
        pltpu.sync_copy(ft_hbm.at[didx], drows)

        def edge(i, carry2):
            svs = []
            lane = lax.iota(jnp.int32, 16)
            ev = jnp.zeros((16,), jnp.float32)
            for hh in range(_H):
                sv = srows[i, pl.ds(hh * _DH, _DH)]
                dv = drows[i, pl.ds(hh * _DH, _DH)]
                svs.append(sv)
                ev = jnp.where(lane == hh, jnp.full((16,), jnp.sum(sv * dv)),
                               ev)
            w = jnp.exp(ev)
            wrows[i, pl.ds(_D, _DH)] = w
            for hh in range(_H):
                wsp = _lane_bcast(w, hh)
                wrows[i, pl.ds(hh * _DH, _DH)] = svs[hh] * wsp
            return carry2

        lax.fori_loop(0, _G, edge, 0)
        pltpu.sync_copy(wrows, acc_sp.at[didx], add=True)
        return carry

    lax.fori_loop(0, _NG, group, 0)
    plsc.subcore_barrier()
    pltpu.sync_copy(acc_sp.at[pl.ds(rbase, _RPS)],
                    acc_hbm.at[pl.ds(c * _NP + rbase, _RPS)])


def _edge(ft, src, dst):
    mesh = plsc.VectorSubcoreMesh(core_axis_name="c", subcore_axis_name="s")
    zeros = jnp.zeros((_NP, _AW), jnp.float32)
    call = pl.kernel(
        _edge_body,
        mesh=mesh,
        compiler_params=pltpu.CompilerParams(needs_layout_passes=False,
                                             use_tc_tiling_on_sc=False),
        out_type=jax.ShapeDtypeStruct((_NC * _NP, _AW), jnp.float32),
        scratch_types=[
            pltpu.VMEM_SHARED((_NP, _AW), jnp.float32),
            pltpu.VMEM((_G,), jnp.int32),
            pltpu.VMEM((_G,), jnp.int32),
            pltpu.VMEM((_G, _D), jnp.float32),
            pltpu.VMEM((_G, _D), jnp.float32),
            pltpu.VMEM((_G, _AW), jnp.float32),
        ],
    )
    return call(ft, src, dst, zeros)


def _elu(x):
    return jnp.where(x > 0, x, jnp.exp(x) - 1.0)


def _post_body(num0_ref, num1_ref, den0_ref, den1_ref, hn_ref, r8_ref,
               hw_ref, hb_ref, ln2w_ref, ln2b_ref,
               w1_ref, b1_ref, w2_ref, b2_ref, out_ref):
    den = den0_ref[...] + den1_ref[...]
    den_e = jnp.dot(den, r8_ref[...], preferred_element_type=jnp.float32)
    num = num0_ref[...] + num1_ref[...]
    rst = jnp.where(den_e > 0, (2.0 * num) / jnp.where(den_e > 0, den_e, 1.0),
                    0.0)
    a = _elu(rst)
    t = jnp.dot(a, hw_ref[...], preferred_element_type=jnp.float32)
    t = t + hb_ref[...] + hn_ref[...]
    mu = jnp.mean(t, axis=1, keepdims=True)
    var = jnp.mean((t - mu) ** 2, axis=1, keepdims=True)
    h3 = (t - mu) * lax.rsqrt(var + 1e-5) * ln2w_ref[...] + ln2b_ref[...]
    f1 = _elu(jnp.dot(h3, w1_ref[...], preferred_element_type=jnp.float32)
              + b1_ref[...])
    f2 = _elu(jnp.dot(f1, w2_ref[...], preferred_element_type=jnp.float32)
              + b2_ref[...])
    out_ref[...] = f2 + h3


def _post(num0, num1, den0, den1, hn, head_W, head_b, ln2_w, ln2_b,
          si_W1, si_b1, si_W2, si_b2):
    blk = 1000
    grid = (_N // blk,)
    r8 = jnp.asarray(np.repeat(np.eye(_H, dtype=np.float32), _DH, axis=1))
    row = lambda i: (i, 0)
    fixed = lambda i: (0, 0)
    return pl.pallas_call(
        _post_body,
        grid=grid,
        in_specs=[
            pl.BlockSpec((blk, _D), row),
            pl.BlockSpec((blk, _D), row),
            pl.BlockSpec((blk, _H), row),
            pl.BlockSpec((blk, _H), row),
            pl.BlockSpec((blk, _D), row),
            pl.BlockSpec((_H, _D), fixed),
            pl.BlockSpec((_D, _D), fixed),
            pl.BlockSpec((1, _D), fixed),
            pl.BlockSpec((1, _D), fixed),
            pl.BlockSpec((1, _D), fixed),
            pl.BlockSpec((_D, 4 * _D), fixed),
            pl.BlockSpec((1, 4 * _D), fixed),
            pl.BlockSpec((4 * _D, _D), fixed),
            pl.BlockSpec((1, _D), fixed),
        ],
        out_specs=pl.BlockSpec((blk, _D), row),
        out_shape=jax.ShapeDtypeStruct((_N, _D), jnp.float32),
    )(num0, num1, den0, den1, hn, r8, head_W, head_b.reshape(1, _D),
      ln2_w.reshape(1, _D), ln2_b.reshape(1, _D),
      si_W1, si_b1.reshape(1, 4 * _D), si_W2, si_b2.reshape(1, _D))


def kernel(h, ln1_w, ln1_b, fc_W, head_W, head_b, ln2_w, ln2_b,
           si_W1, si_b1, si_W2, si_b2, edge_index):
    hn, ft = _pre(h, ln1_w, ln1_b, fc_W)
    src = edge_index[0]
    dst = edge_index[1]
    acc = _edge(ft, src, dst)
    num0 = acc[:_N, :_D]
    num1 = acc[_NP:_NP + _N, :_D]
    den0 = acc[:_N, _D:_D + _H]
    den1 = acc[_NP:_NP + _N, _D:_D + _H]
    return _post(num0, num1, den0, den1, hn, head_W, head_b, ln2_w, ln2_b,
                 si_W1, si_b1, si_W2, si_b2)


# speedup vs baseline: 76.4093x; 1.5549x over previous
import functools

import jax
import jax.numpy as jnp
import numpy as np
from jax import lax
from jax.experimental import pallas as pl
from jax.experimental.pallas import tpu as pltpu
from jax.experimental.pallas import tpu_sc as plsc

_N = 10000
_E = 320000
_D = 128
_H = 8
_DH = 16

_NC = 2
_NS = 16
_NW = _NC * _NS
_EW = _E // _NW
_G = 40
_NG = _EW // _G
_NP = 10240
_RPS = _NP // _NS
_AW = 144


def _pre_body(h_ref, w_ref, b_ref, fc_ref, hn_ref, ft_ref):
    x = h_ref[...]
    mu = jnp.mean(x, axis=1, keepdims=True)
    var = jnp.mean((x - mu) ** 2, axis=1, keepdims=True)
    hn = (x - mu) * lax.rsqrt(var + 1e-5) * w_ref[...] + b_ref[...]
    hn_ref[...] = hn
    ft_ref[...] = 0.5 * jnp.dot(hn, fc_ref[...],
                                preferred_element_type=jnp.float32)


def _pre(h, ln1_w, ln1_b, fc_W):
    blk = 1000
    grid = (_N // blk,)
    return pl.pallas_call(
        _pre_body,
        grid=grid,
        in_specs=[
            pl.BlockSpec((blk, _D), lambda i: (i, 0)),
            pl.BlockSpec((1, _D), lambda i: (0, 0)),
            pl.BlockSpec((1, _D), lambda i: (0, 0)),
            pl.BlockSpec((_D, _D), lambda i: (0, 0)),
        ],
        out_specs=[
            pl.BlockSpec((blk, _D), lambda i: (i, 0)),
            pl.BlockSpec((blk, _D), lambda i: (i, 0)),
        ],
        out_shape=[
            jax.ShapeDtypeStruct((_N, _D), jnp.float32),
            jax.ShapeDtypeStruct((_N, _D), jnp.float32),
        ],
    )(h, ln1_w.reshape(1, _D), ln1_b.reshape(1, _D), fc_W)


def _lane_bcast(v, lane):
    idx = jnp.full((16, 1), lane, jnp.int32)
    dn = lax.GatherDimensionNumbers(offset_dims=(), collapsed_slice_dims=(0,),
                                    start_index_map=(0,))
    return lax.gather(v, idx, dn, (1,),
                      mode=lax.GatherScatterMode.PROMISE_IN_BOUNDS)


_NB = 3


def _edge_body(ft_hbm, eidx_hbm, zn_hbm, zd_hbm, num_hbm, den_hbm,
               num_sp, den_sp, ibuf, srows, drows, wden, rsem, ssem):
    c = lax.axis_index("c")
    s = lax.axis_index("s")
    rbase = s * _RPS
    pltpu.sync_copy(zn_hbm.at[pl.ds(rbase, _RPS)],
                    num_sp.at[pl.ds(rbase, _RPS)])
    pltpu.sync_copy(zd_hbm.at[pl.ds(rbase, _RPS)],
                    den_sp.at[pl.ds(rbase, _RPS)])
    plsc.subcore_barrier()

    wid = c * _NS + s
    gbase = wid * _NG

    def fill(g, b):
        pltpu.sync_copy(eidx_hbm.at[gbase + g], ibuf[b])
        pltpu.async_copy(ft_hbm.at[ibuf[b].at[0]], srows[b], rsem.at[b])
        pltpu.async_copy(ft_hbm.at[ibuf[b].at[1]], drows[b], rsem.at[b])

    def wait_rows(b):
        pltpu.make_async_copy(ft_hbm.at[ibuf[b].at[0]], srows[b],
                              rsem.at[b]).wait()
        pltpu.make_async_copy(ft_hbm.at[ibuf[b].at[1]], drows[b],
                              rsem.at[b]).wait()

    def scatter(b):
        pltpu.async_copy(srows[b], num_sp.at[ibuf[b].at[1]], ssem.at[b],
                         add=True)
        pltpu.async_copy(wden[b], den_sp.at[ibuf[b].at[1]], ssem.at[b],
                         add=True)

    def wait_scatter(b):
        pltpu.make_async_copy(srows[b], num_sp.at[ibuf[b].at[1]],
                              ssem.at[b]).wait()
        pltpu.make_async_copy(wden[b], den_sp.at[ibuf[b].at[1]],
                              ssem.at[b]).wait()

    def compute(b):
        def edge(i, carry):
            svs = []
            lane = lax.iota(jnp.int32, 16)
            ev = jnp.zeros((16,), jnp.float32)
            for hh in range(_H):
                sv = srows[b][i, pl.ds(hh * _DH, _DH)]
                dv = drows[b][i, pl.ds(hh * _DH, _DH)]
                svs.append(sv)
                ev = jnp.where(lane == hh, jnp.full((16,), jnp.sum(sv * dv)),
                               ev)
            w = jnp.exp(ev)
            wden[b][i, :] = w
            for hh in range(_H):
                srows[b][i, pl.ds(hh * _DH, _DH)] = svs[hh] * _lane_bcast(w, hh)
            return carry

        lax.fori_loop(0, _G, edge, 0, unroll=2)

    for k in range(2):
        fill(k, k)

    nmain = (_NG - 4) // _NB

    def step(t, carry):
        for off in range(_NB):
            g = _NB * t + off
            bn = (off + 2) % _NB
            wait_rows(off)
            compute(off)
            scatter(off)
            @pl.when(g >= 1)
            def _():
                wait_scatter(bn)
            fill(g + 2, bn)
        return carry

    lax.fori_loop(0, nmain, step, 0)

    for g in range(_NB * nmain, _NG):
        b = g % _NB
        wait_rows(b)
        compute(b)
        scatter(b)
        wait_scatter((b + 2) % _NB)
        if g + 2 < _NG:
            fill(g + 2, (g + 2) % _NB)
    wait_scatter((_NG - 1) % _NB)

    plsc.subcore_barrier()
    pltpu.sync_copy(num_sp.at[pl.ds(rbase, _RPS)],
                    num_hbm.at[pl.ds(c * _NP + rbase, _RPS)])
    pltpu.sync_copy(den_sp.at[pl.ds(rbase, _RPS)],
                    den_hbm.at[pl.ds(c * _NP + rbase, _RPS)])


def _edge(ft, eidx2):
    mesh = plsc.VectorSubcoreMesh(core_axis_name="c", subcore_axis_name="s")
    zn = jnp.zeros((_NP, _D), jnp.float32)
    zd = jnp.zeros((_NP, _DH), jnp.float32)
    call = pl.kernel(
        _edge_body,
        mesh=mesh,
        compiler_params=pltpu.CompilerParams(needs_layout_passes=False,
                                             use_tc_tiling_on_sc=False),
        out_type=[
            jax.ShapeDtypeStruct((_NC * _NP, _D), jnp.float32),
            jax.ShapeDtypeStruct((_NC * _NP, _DH), jnp.float32),
        ],
        scratch_types=[
            pltpu.VMEM_SHARED((_NP, _D), jnp.float32),
            pltpu.VMEM_SHARED((_NP, _DH), jnp.float32),
            [pltpu.VMEM((2, _G), jnp.int32) for _ in range(_NB)],
            [pltpu.VMEM((_G, _D), jnp.float32) for _ in range(_NB)],
            [pltpu.VMEM((_G, _D), jnp.float32) for _ in range(_NB)],
            [pltpu.VMEM((_G, _DH), jnp.float32) for _ in range(_NB)],
            pltpu.SemaphoreType.DMA((_NB,)),
            pltpu.SemaphoreType.DMA((_NB,)),
        ],
    )
    return call(ft, eidx2, zn, zd)


def _elu(x):
    return jnp.where(x > 0, x, jnp.exp(x) - 1.0)


def _post_body(num0_ref, num1_ref, den0_ref, den1_ref, hn_ref, r8_ref,
               hw_ref, hb_ref, ln2w_ref, ln2b_ref,
               w1_ref, b1_ref, w2_ref, b2_ref, out_ref):
    den = den0_ref[...] + den1_ref[...]
    den_e = jnp.dot(den, r8_ref[...], preferred_element_type=jnp.float32)
    num = num0_ref[...] + num1_ref[...]
    rst = jnp.where(den_e > 0, (2.0 * num) / jnp.where(den_e > 0, den_e, 1.0),
                    0.0)
    a = _elu(rst)
    t = jnp.dot(a, hw_ref[...], preferred_element_type=jnp.float32)
    t = t + hb_ref[...] + hn_ref[...]
    mu = jnp.mean(t, axis=1, keepdims=True)
    var = jnp.mean((t - mu) ** 2, axis=1, keepdims=True)
    h3 = (t - mu) * lax.rsqrt(var + 1e-5) * ln2w_ref[...] + ln2b_ref[...]
    f1 = _elu(jnp.dot(h3, w1_ref[...], preferred_element_type=jnp.float32)
              + b1_ref[...])
    f2 = _elu(jnp.dot(f1, w2_ref[...], preferred_element_type=jnp.float32)
              + b2_ref[...])
    out_ref[...] = f2 + h3


def _post(num0, num1, den0, den1, hn, head_W, head_b, ln2_w, ln2_b,
          si_W1, si_b1, si_W2, si_b2):
    blk = 1000
    grid = (_N // blk,)
    r8 = jnp.asarray(np.repeat(np.eye(_H, dtype=np.float32), _DH, axis=1))
    row = lambda i: (i, 0)
    fixed = lambda i: (0, 0)
    return pl.pallas_call(
        _post_body,
        grid=grid,
        in_specs=[
            pl.BlockSpec((blk, _D), row),
            pl.BlockSpec((blk, _D), row),
            pl.BlockSpec((blk, _H), row),
            pl.BlockSpec((blk, _H), row),
            pl.BlockSpec((blk, _D), row),
            pl.BlockSpec((_H, _D), fixed),
            pl.BlockSpec((_D, _D), fixed),
            pl.BlockSpec((1, _D), fixed),
            pl.BlockSpec((1, _D), fixed),
            pl.BlockSpec((1, _D), fixed),
            pl.BlockSpec((_D, 4 * _D), fixed),
            pl.BlockSpec((1, 4 * _D), fixed),
            pl.BlockSpec((4 * _D, _D), fixed),
            pl.BlockSpec((1, _D), fixed),
        ],
        out_specs=pl.BlockSpec((blk, _D), row),
        out_shape=jax.ShapeDtypeStruct((_N, _D), jnp.float32),
    )(num0, num1, den0, den1, hn, r8, head_W, head_b.reshape(1, _D),
      ln2_w.reshape(1, _D), ln2_b.reshape(1, _D),
      si_W1, si_b1.reshape(1, 4 * _D), si_W2, si_b2.reshape(1, _D))


def kernel(h, ln1_w, ln1_b, fc_W, head_W, head_b, ln2_w, ln2_b,
           si_W1, si_b1, si_W2, si_b2, edge_index):
    hn, ft = _pre(h, ln1_w, ln1_b, fc_W)
    eidx2 = jnp.stack(
        [edge_index[0].reshape(_NW, _NG, _G),
         edge_index[1].reshape(_NW, _NG, _G)], axis=2,
    ).reshape(_NW * _NG, 2, _G)
    num, den = _edge(ft, eidx2)
    num0 = num[:_N]
    num1 = num[_NP:_NP + _N]
    den0 = den[:_N, :_H]
    den1 = den[_NP:_NP + _N, :_H]
    return _post(num0, num1, den0, den1, hn, head_W, head_b, ln2_w, ln2_b,
                 si_W1, si_b1, si_W2, si_b2)
